# baseline (speedup 1.0000x reference)
import functools

import jax
import jax.numpy as jnp
from jax import lax
from jax.experimental import pallas as pl
from jax.experimental.pallas import tpu as pltpu
from jax.experimental.pallas import tpu_sc as plsc

_info = plsc.get_sparse_core_info()
_NC = _info.num_cores
_NS = _info.num_subcores
_NW = _NC * _NS

_R = 2
_TW = 512


def _transpose_body(tt_ref, o_ref):
    t = tt_ref[...]
    d = t.shape[0]
    eye = jnp.eye(d, dtype=t.dtype)
    tr = lax.dot_general(t, eye, (((0,), (0,)), ((), ())),
                         preferred_element_type=jnp.float32,
                         precision=lax.Precision.HIGHEST)
    o_ref[:, pl.ds(0, d)] = tr


@functools.lru_cache(maxsize=None)
def _make_transpose(V, D):
    grid = (V + _TW - 1) // _TW
    return pl.pallas_call(
        _transpose_body,
        grid=(grid,),
        in_specs=[pl.BlockSpec((D, _TW), lambda i: (0, i))],
        out_specs=pl.BlockSpec((_TW, 128), lambda i: (i, 0)),
        out_shape=jax.ShapeDtypeStruct((V, 128), jnp.float32),
    )


@functools.lru_cache(maxsize=None)
def _make_gather(B, H, V, D):
    assert B % (_NW * 2 * _R) == 0
    rows_per_w = B // _NW
    n_idx_w = rows_per_w * H
    n_chunks = rows_per_w // _R
    n_groups = n_chunks // 2
    mesh = plsc.VectorSubcoreMesh(core_axis_name="c", subcore_axis_name="s")

    @functools.partial(
        pl.kernel,
        mesh=mesh,
        compiler_params=pltpu.CompilerParams(use_tc_tiling_on_sc=False),
        out_type=jax.ShapeDtypeStruct((B * H, 2 * D), jnp.float32),
        scratch_types=[
            pltpu.VMEM((n_idx_w,), jnp.int32),
            pltpu.VMEM((_R * H, D), jnp.float32),
            pltpu.VMEM((_R * H, D), jnp.float32),
            pltpu.SemaphoreType.DMA,
            pltpu.SemaphoreType.DMA,
            pltpu.SemaphoreType.DMA,
            pltpu.SemaphoreType.DMA,
        ],
    )
    def gather_kernel(x_hbm, table_hbm, out_hbm, idx_all, rows0, rows1,
                      gsem0, gsem1, osem0, osem1):
        wid = lax.axis_index("s") * _NC + lax.axis_index("c")
        base = wid * rows_per_w
        pltpu.sync_copy(x_hbm.at[pl.ds(base * H, n_idx_w)], idx_all)

        def gather(c, buf, sem):
            return pltpu.make_async_copy(
                table_hbm.at[idx_all.at[pl.ds(c * _R * H, _R * H)]], buf, sem)

        def wback(c, buf, sem):
            r0 = (base + c * _R) * H
            return pltpu.make_async_copy(
                buf, out_hbm.at[pl.ds(r0, _R * H), pl.ds(0, D)], sem)

        gather(0, rows0, gsem0).start()

        def body(g, carry):
            c0 = 2 * g
            c1 = c0 + 1
            gather(c0, rows0, gsem0).wait()

            @pl.when(g > 0)
            def _():
                wback(c1 - 2, rows1, osem1).wait()

            gather(c1, rows1, gsem1).start()
            wback(c0, rows0, osem0).start()
            gather(c1, rows1, gsem1).wait()
            wback(c0, rows0, osem0).wait()

            @pl.when(g < n_groups - 1)
            def _():
                gather(c0 + 2, rows0, gsem0).start()

            wback(c1, rows1, osem1).start()
            return carry

        lax.fori_loop(0, n_groups, body, 0)
        wback(n_chunks - 1, rows1, osem1).wait()

    return gather_kernel


def kernel(x, table):
    V, D = table.shape
    B, H = x.shape
    tp = _make_transpose(V, D)(table.T).reshape(2 * V, D)
    x2 = (x.astype(jnp.int32) * 2).reshape(B * H)
    out = _make_gather(B, H, V, D)(x2, tp)
    return out.reshape(B, H, 2 * D)[:, :, :D]


# speedup vs baseline: 2.1214x; 2.1214x over previous
import functools

import jax
import jax.numpy as jnp
from jax import lax
from jax.experimental import pallas as pl
from jax.experimental.pallas import tpu as pltpu
from jax.experimental.pallas import tpu_sc as plsc

_info = plsc.get_sparse_core_info()
_NC = _info.num_cores
_NS = _info.num_subcores
_NW = _NC * _NS

_R = 2
_TW = 4096


def _transpose_body(tt_ref, o_ref):
    t = tt_ref[...]
    d = t.shape[0]
    eye = jnp.eye(d, dtype=t.dtype)
    tr = lax.dot_general(t, eye, (((0,), (0,)), ((), ())),
                         preferred_element_type=jnp.float32,
                         precision=lax.Precision.HIGHEST)
    o_ref[:, pl.ds(0, d)] = tr


@functools.lru_cache(maxsize=None)
def _make_transpose(V, D):
    grid = (V + _TW - 1) // _TW
    return pl.pallas_call(
        _transpose_body,
        grid=(grid,),
        in_specs=[pl.BlockSpec((D, _TW), lambda i: (0, i))],
        out_specs=pl.BlockSpec((_TW, 128), lambda i: (i, 0)),
        out_shape=jax.ShapeDtypeStruct((V, 128), jnp.float32),
    )


@functools.lru_cache(maxsize=None)
def _make_gather(B, H, V, D):
    assert B % (_NW * 2 * _R) == 0
    rows_per_w = B // _NW
    n_idx_w = rows_per_w * H
    n_chunks = rows_per_w // _R
    n_groups = n_chunks // 2
    mesh = plsc.VectorSubcoreMesh(core_axis_name="c", subcore_axis_name="s")

    @functools.partial(
        pl.kernel,
        mesh=mesh,
        compiler_params=pltpu.CompilerParams(use_tc_tiling_on_sc=False),
        out_type=jax.ShapeDtypeStruct((B * H, 2 * D), jnp.float32),
        scratch_types=[
            pltpu.VMEM((n_idx_w,), jnp.int32),
            pltpu.VMEM((_R * H, D), jnp.float32),
            pltpu.VMEM((_R * H, D), jnp.float32),
            pltpu.SemaphoreType.DMA,
            pltpu.SemaphoreType.DMA,
            pltpu.SemaphoreType.DMA,
            pltpu.SemaphoreType.DMA,
        ],
    )
    def gather_kernel(x_hbm, table_hbm, out_hbm, idx_all, rows0, rows1,
                      gsem0, gsem1, osem0, osem1):
        wid = lax.axis_index("s") * _NC + lax.axis_index("c")
        base = wid * rows_per_w
        pltpu.sync_copy(x_hbm.at[pl.ds(base * H, n_idx_w)], idx_all)

        def gather(c, buf, sem):
            return pltpu.make_async_copy(
                table_hbm.at[idx_all.at[pl.ds(c * _R * H, _R * H)]], buf, sem)

        def wback(c, buf, sem):
            r0 = (base + c * _R) * H
            return pltpu.make_async_copy(
                buf, out_hbm.at[pl.ds(r0, _R * H), pl.ds(0, D)], sem)

        gather(0, rows0, gsem0).start()

        def body(g, carry):
            c0 = 2 * g
            c1 = c0 + 1
            gather(c0, rows0, gsem0).wait()

            @pl.when(g > 0)
            def _():
                wback(c1 - 2, rows1, osem1).wait()

            gather(c1, rows1, gsem1).start()
            wback(c0, rows0, osem0).start()
            gather(c1, rows1, gsem1).wait()
            wback(c0, rows0, osem0).wait()

            @pl.when(g < n_groups - 1)
            def _():
                gather(c0 + 2, rows0, gsem0).start()

            wback(c1, rows1, osem1).start()
            return carry

        lax.fori_loop(0, n_groups, body, 0)
        wback(n_chunks - 1, rows1, osem1).wait()

    return gather_kernel


def kernel(x, table):
    V, D = table.shape
    B, H = x.shape
    tp = _make_transpose(V, D)(table.T).reshape(2 * V, D)
    x2 = (x.astype(jnp.int32) * 2).reshape(B * H)
    out = _make_gather(B, H, V, D)(x2, tp)
    return out.reshape(B, H, 2 * D)[:, :, :D]


# speedup vs baseline: 2.2333x; 1.0528x over previous
import functools

import jax
import jax.numpy as jnp
from jax import lax
from jax.experimental import pallas as pl
from jax.experimental.pallas import tpu as pltpu
from jax.experimental.pallas import tpu_sc as plsc

_info = plsc.get_sparse_core_info()
_NC = _info.num_cores
_NS = _info.num_subcores
_NW = _NC * _NS

_R = 2
_TW = 8192


def _transpose_body(tt_ref, o_ref):
    t = tt_ref[...]
    d = t.shape[0]
    eye = jnp.eye(d, dtype=t.dtype)
    tr = lax.dot_general(t, eye, (((0,), (0,)), ((), ())),
                         preferred_element_type=jnp.float32,
                         precision=lax.Precision.HIGHEST)
    o_ref[:, pl.ds(0, d)] = tr


@functools.lru_cache(maxsize=None)
def _make_transpose(V, D):
    grid = (V + _TW - 1) // _TW
    return pl.pallas_call(
        _transpose_body,
        grid=(grid,),
        in_specs=[pl.BlockSpec((D, _TW), lambda i: (0, i))],
        out_specs=pl.BlockSpec((_TW, 128), lambda i: (i, 0)),
        out_shape=jax.ShapeDtypeStruct((V, 128), jnp.float32),
    )


@functools.lru_cache(maxsize=None)
def _make_gather(B, H, V, D):
    assert B % (_NW * 2 * _R) == 0
    rows_per_w = B // _NW
    n_idx_w = rows_per_w * H
    n_chunks = rows_per_w // _R
    n_groups = n_chunks // 2
    mesh = plsc.VectorSubcoreMesh(core_axis_name="c", subcore_axis_name="s")

    @functools.partial(
        pl.kernel,
        mesh=mesh,
        compiler_params=pltpu.CompilerParams(use_tc_tiling_on_sc=False),
        out_type=jax.ShapeDtypeStruct((B * H, 2 * D), jnp.float32),
        scratch_types=[
            pltpu.VMEM((n_idx_w,), jnp.int32),
            pltpu.VMEM((_R * H, D), jnp.float32),
            pltpu.VMEM((_R * H, D), jnp.float32),
            pltpu.SemaphoreType.DMA,
            pltpu.SemaphoreType.DMA,
            pltpu.SemaphoreType.DMA,
            pltpu.SemaphoreType.DMA,
        ],
    )
    def gather_kernel(x_hbm, table_hbm, out_hbm, idx_all, rows0, rows1,
                      gsem0, gsem1, osem0, osem1):
        wid = lax.axis_index("s") * _NC + lax.axis_index("c")
        base = wid * rows_per_w
        pltpu.sync_copy(x_hbm.at[pl.ds(base * H, n_idx_w)], idx_all)

        def gather(c, buf, sem):
            return pltpu.make_async_copy(
                table_hbm.at[idx_all.at[pl.ds(c * _R * H, _R * H)]], buf, sem)

        def wback(c, buf, sem):
            r0 = (base + c * _R) * H
            return pltpu.make_async_copy(
                buf, out_hbm.at[pl.ds(r0, _R * H), pl.ds(0, D)], sem)

        gather(0, rows0, gsem0).start()

        def body(g, carry):
            c0 = 2 * g
            c1 = c0 + 1
            gather(c0, rows0, gsem0).wait()

            @pl.when(g > 0)
            def _():
                wback(c1 - 2, rows1, osem1).wait()

            gather(c1, rows1, gsem1).start()
            wback(c0, rows0, osem0).start()
            gather(c1, rows1, gsem1).wait()
            wback(c0, rows0, osem0).wait()

            @pl.when(g < n_groups - 1)
            def _():
                gather(c0 + 2, rows0, gsem0).start()

            wback(c1, rows1, osem1).start()
            return carry

        lax.fori_loop(0, n_groups, body, 0)
        wback(n_chunks - 1, rows1, osem1).wait()

    return gather_kernel


def kernel(x, table):
    V, D = table.shape
    B, H = x.shape
    tp = _make_transpose(V, D)(table.T).reshape(2 * V, D)
    x2 = (x.astype(jnp.int32) * 2).reshape(B * H)
    out = _make_gather(B, H, V, D)(x2, tp)
    return out.reshape(B, H, 2 * D)[:, :, :D]


# speedup vs baseline: 2.2607x; 1.0123x over previous
import functools

import jax
import jax.numpy as jnp
from jax import lax
from jax.experimental import pallas as pl
from jax.experimental.pallas import tpu as pltpu
from jax.experimental.pallas import tpu_sc as plsc

_info = plsc.get_sparse_core_info()
_NC = _info.num_cores
_NS = _info.num_subcores
_NW = _NC * _NS

_R = 2
_TW = 12288


def _transpose_body(tt_ref, o_ref):
    t = tt_ref[...]
    d = t.shape[0]
    eye = jnp.eye(d, dtype=t.dtype)
    tr = lax.dot_general(t, eye, (((0,), (0,)), ((), ())),
                         preferred_element_type=jnp.float32,
                         precision=lax.Precision.HIGHEST)
    o_ref[:, pl.ds(0, d)] = tr


@functools.lru_cache(maxsize=None)
def _make_transpose(V, D):
    grid = (V + _TW - 1) // _TW
    return pl.pallas_call(
        _transpose_body,
        grid=(grid,),
        in_specs=[pl.BlockSpec((D, _TW), lambda i: (0, i))],
        out_specs=pl.BlockSpec((_TW, 128), lambda i: (i, 0)),
        out_shape=jax.ShapeDtypeStruct((V, 128), jnp.float32),
    )


@functools.lru_cache(maxsize=None)
def _make_gather(B, H, V, D):
    assert B % (_NW * 2 * _R) == 0
    rows_per_w = B // _NW
    n_idx_w = rows_per_w * H
    n_chunks = rows_per_w // _R
    n_groups = n_chunks // 2
    mesh = plsc.VectorSubcoreMesh(core_axis_name="c", subcore_axis_name="s")

    @functools.partial(
        pl.kernel,
        mesh=mesh,
        compiler_params=pltpu.CompilerParams(use_tc_tiling_on_sc=False),
        out_type=jax.ShapeDtypeStruct((B * H, 2 * D), jnp.float32),
        scratch_types=[
            pltpu.VMEM((n_idx_w,), jnp.int32),
            pltpu.VMEM((_R * H, D), jnp.float32),
            pltpu.VMEM((_R * H, D), jnp.float32),
            pltpu.SemaphoreType.DMA,
            pltpu.SemaphoreType.DMA,
            pltpu.SemaphoreType.DMA,
            pltpu.SemaphoreType.DMA,
        ],
    )
    def gather_kernel(x_hbm, table_hbm, out_hbm, idx_all, rows0, rows1,
                      gsem0, gsem1, osem0, osem1):
        wid = lax.axis_index("s") * _NC + lax.axis_index("c")
        base = wid * rows_per_w
        pltpu.sync_copy(x_hbm.at[pl.ds(base * H, n_idx_w)], idx_all)

        def gather(c, buf, sem):
            return pltpu.make_async_copy(
                table_hbm.at[idx_all.at[pl.ds(c * _R * H, _R * H)]], buf, sem)

        def wback(c, buf, sem):
            r0 = (base + c * _R) * H
            return pltpu.make_async_copy(
                buf, out_hbm.at[pl.ds(r0, _R * H), pl.ds(0, D)], sem)

        gather(0, rows0, gsem0).start()

        def body(g, carry):
            c0 = 2 * g
            c1 = c0 + 1
            gather(c0, rows0, gsem0).wait()

            @pl.when(g > 0)
            def _():
                wback(c1 - 2, rows1, osem1).wait()

            gather(c1, rows1, gsem1).start()
            wback(c0, rows0, osem0).start()
            gather(c1, rows1, gsem1).wait()
            wback(c0, rows0, osem0).wait()

            @pl.when(g < n_groups - 1)
            def _():
                gather(c0 + 2, rows0, gsem0).start()

            wback(c1, rows1, osem1).start()
            return carry

        lax.fori_loop(0, n_groups, body, 0)
        wback(n_chunks - 1, rows1, osem1).wait()

    return gather_kernel


def kernel(x, table):
    V, D = table.shape
    B, H = x.shape
    tp = _make_transpose(V, D)(table.T).reshape(2 * V, D)
    x2 = (x.astype(jnp.int32) * 2).reshape(B * H)
    out = _make_gather(B, H, V, D)(x2, tp)
    return out.reshape(B, H, 2 * D)[:, :, :D]


# speedup vs baseline: 2.2636x; 1.0012x over previous
import functools

import jax
import jax.numpy as jnp
from jax import lax
from jax.experimental import pallas as pl
from jax.experimental.pallas import tpu as pltpu
from jax.experimental.pallas import tpu_sc as plsc

_info = plsc.get_sparse_core_info()
_NC = _info.num_cores
_NS = _info.num_subcores
_NW = _NC * _NS

_R = 4
_TW = 12288


def _transpose_body(tt_ref, o_ref):
    t = tt_ref[...]
    d = t.shape[0]
    eye = jnp.eye(d, dtype=t.dtype)
    tr = lax.dot_general(t, eye, (((0,), (0,)), ((), ())),
                         preferred_element_type=jnp.float32,
                         precision=lax.Precision.HIGHEST)
    o_ref[:, pl.ds(0, d)] = tr


@functools.lru_cache(maxsize=None)
def _make_transpose(V, D):
    grid = (V + _TW - 1) // _TW
    return pl.pallas_call(
        _transpose_body,
        grid=(grid,),
        in_specs=[pl.BlockSpec((D, _TW), lambda i: (0, i))],
        out_specs=pl.BlockSpec((_TW, 128), lambda i: (i, 0)),
        out_shape=jax.ShapeDtypeStruct((V, 128), jnp.float32),
    )


@functools.lru_cache(maxsize=None)
def _make_gather(B, H, V, D):
    assert B % (_NW * 2 * _R) == 0
    rows_per_w = B // _NW
    n_idx_w = rows_per_w * H
    n_chunks = rows_per_w // _R
    n_groups = n_chunks // 2
    mesh = plsc.VectorSubcoreMesh(core_axis_name="c", subcore_axis_name="s")

    @functools.partial(
        pl.kernel,
        mesh=mesh,
        compiler_params=pltpu.CompilerParams(use_tc_tiling_on_sc=False),
        out_type=jax.ShapeDtypeStruct((B * H, 2 * D), jnp.float32),
        scratch_types=[
            pltpu.VMEM((n_idx_w,), jnp.int32),
            pltpu.VMEM((_R * H, D), jnp.float32),
            pltpu.VMEM((_R * H, D), jnp.float32),
            pltpu.SemaphoreType.DMA,
            pltpu.SemaphoreType.DMA,
            pltpu.SemaphoreType.DMA,
            pltpu.SemaphoreType.DMA,
        ],
    )
    def gather_kernel(x_hbm, table_hbm, out_hbm, idx_all, rows0, rows1,
                      gsem0, gsem1, osem0, osem1):
        wid = lax.axis_index("s") * _NC + lax.axis_index("c")
        base = wid * rows_per_w
        pltpu.sync_copy(x_hbm.at[pl.ds(base * H, n_idx_w)], idx_all)

        def gather(c, buf, sem):
            return pltpu.make_async_copy(
                table_hbm.at[idx_all.at[pl.ds(c * _R * H, _R * H)]], buf, sem)

        def wback(c, buf, sem):
            r0 = (base + c * _R) * H
            return pltpu.make_async_copy(
                buf, out_hbm.at[pl.ds(r0, _R * H), pl.ds(0, D)], sem)

        gather(0, rows0, gsem0).start()

        def body(g, carry):
            c0 = 2 * g
            c1 = c0 + 1
            gather(c0, rows0, gsem0).wait()

            @pl.when(g > 0)
            def _():
                wback(c1 - 2, rows1, osem1).wait()

            gather(c1, rows1, gsem1).start()
            wback(c0, rows0, osem0).start()
            gather(c1, rows1, gsem1).wait()
            wback(c0, rows0, osem0).wait()

            @pl.when(g < n_groups - 1)
            def _():
                gather(c0 + 2, rows0, gsem0).start()

            wback(c1, rows1, osem1).start()
            return carry

        lax.fori_loop(0, n_groups, body, 0)
        wback(n_chunks - 1, rows1, osem1).wait()

    return gather_kernel


def kernel(x, table):
    V, D = table.shape
    B, H = x.shape
    tp = _make_transpose(V, D)(table.T).reshape(2 * V, D)
    x2 = (x.astype(jnp.int32) * 2).reshape(B * H)
    out = _make_gather(B, H, V, D)(x2, tp)
    return out.reshape(B, H, 2 * D)[:, :, :D]


# speedup vs baseline: 2.8252x; 1.2481x over previous
import functools

import jax
import jax.numpy as jnp
from jax import lax
from jax.experimental import pallas as pl
from jax.experimental.pallas import tpu as pltpu
from jax.experimental.pallas import tpu_sc as plsc

_info = plsc.get_sparse_core_info()
_NC = _info.num_cores
_NS = _info.num_subcores
_NW = _NC * _NS

_R = 4
_TW = 12288


def _transpose_body(tt_ref, o_ref):
    t = tt_ref[...]
    d = t.shape[0]
    eye = jnp.eye(d, dtype=jnp.bfloat16)
    b1 = t.astype(jnp.bfloat16)
    r1 = t - b1.astype(jnp.float32)
    b2 = r1.astype(jnp.bfloat16)
    b3 = (r1 - b2.astype(jnp.float32)).astype(jnp.bfloat16)
    dn = (((0,), (0,)), ((), ()))
    tr = (lax.dot_general(b1, eye, dn, preferred_element_type=jnp.float32)
          + lax.dot_general(b2, eye, dn, preferred_element_type=jnp.float32)
          + lax.dot_general(b3, eye, dn, preferred_element_type=jnp.float32))
    o_ref[:, pl.ds(0, d)] = tr


@functools.lru_cache(maxsize=None)
def _make_transpose(V, D):
    grid = (V + _TW - 1) // _TW
    return pl.pallas_call(
        _transpose_body,
        grid=(grid,),
        in_specs=[pl.BlockSpec((D, _TW), lambda i: (0, i))],
        out_specs=pl.BlockSpec((_TW, 128), lambda i: (i, 0)),
        out_shape=jax.ShapeDtypeStruct((V, 128), jnp.float32),
    )


@functools.lru_cache(maxsize=None)
def _make_gather(B, H, V, D):
    assert B % (_NW * 2 * _R) == 0
    rows_per_w = B // _NW
    n_idx_w = rows_per_w * H
    n_chunks = rows_per_w // _R
    n_groups = n_chunks // 2
    mesh = plsc.VectorSubcoreMesh(core_axis_name="c", subcore_axis_name="s")

    @functools.partial(
        pl.kernel,
        mesh=mesh,
        compiler_params=pltpu.CompilerParams(use_tc_tiling_on_sc=False),
        out_type=jax.ShapeDtypeStruct((B * H, 2 * D), jnp.float32),
        scratch_types=[
            pltpu.VMEM((n_idx_w,), jnp.int32),
            pltpu.VMEM((_R * H, D), jnp.float32),
            pltpu.VMEM((_R * H, D), jnp.float32),
            pltpu.SemaphoreType.DMA,
            pltpu.SemaphoreType.DMA,
            pltpu.SemaphoreType.DMA,
            pltpu.SemaphoreType.DMA,
        ],
    )
    def gather_kernel(x_hbm, table_hbm, out_hbm, idx_all, rows0, rows1,
                      gsem0, gsem1, osem0, osem1):
        wid = lax.axis_index("s") * _NC + lax.axis_index("c")
        base = wid * rows_per_w
        pltpu.sync_copy(x_hbm.at[pl.ds(base * H, n_idx_w)], idx_all)

        def gather(c, buf, sem):
            return pltpu.make_async_copy(
                table_hbm.at[idx_all.at[pl.ds(c * _R * H, _R * H)]], buf, sem)

        def wback(c, buf, sem):
            r0 = (base + c * _R) * H
            return pltpu.make_async_copy(
                buf, out_hbm.at[pl.ds(r0, _R * H), pl.ds(0, D)], sem)

        gather(0, rows0, gsem0).start()

        def body(g, carry):
            c0 = 2 * g
            c1 = c0 + 1
            gather(c0, rows0, gsem0).wait()

            @pl.when(g > 0)
            def _():
                wback(c1 - 2, rows1, osem1).wait()

            gather(c1, rows1, gsem1).start()
            wback(c0, rows0, osem0).start()
            gather(c1, rows1, gsem1).wait()
            wback(c0, rows0, osem0).wait()

            @pl.when(g < n_groups - 1)
            def _():
                gather(c0 + 2, rows0, gsem0).start()

            wback(c1, rows1, osem1).start()
            return carry

        lax.fori_loop(0, n_groups, body, 0)
        wback(n_chunks - 1, rows1, osem1).wait()

    return gather_kernel


def kernel(x, table):
    V, D = table.shape
    B, H = x.shape
    tp = _make_transpose(V, D)(table.T).reshape(2 * V, D)
    x2 = (x.astype(jnp.int32) * 2).reshape(B * H)
    out = _make_gather(B, H, V, D)(x2, tp)
    return out.reshape(B, H, 2 * D)[:, :, :D]
